# Initial kernel scaffold; baseline (speedup 1.0000x reference)
import jax
import jax.numpy as jnp
from jax.experimental import pallas as pl


def kernel(payload_tensor, W, b):
    raise NotImplementedError("write your pallas kernel here")


# speedup vs baseline: 4.9886x; 4.9886x over previous
import functools

import jax
import jax.numpy as jnp
from jax import lax
from jax.experimental import pallas as pl
from jax.experimental.pallas import tpu as pltpu

TOKENS = 32768
D_MODEL = 768
NUM_CHILDREN = 64
TOP_K = 2

BT = 512


def _fused_body(x_ref, w_ref, b_ref, o_ref):
    logits = lax.dot_general(
        x_ref[...], w_ref[...],
        (((1,), (1,)), ((), ())),
        preferred_element_type=jnp.float32,
    ) + b_ref[...]
    e = lax.broadcasted_iota(jnp.int32, logits.shape, 1)
    m1 = jnp.max(logits, axis=1, keepdims=True)
    i1 = jnp.min(jnp.where(logits == m1, e, NUM_CHILDREN), axis=1, keepdims=True)
    lx = jnp.where(e == i1, -jnp.inf, logits)
    m2 = jnp.max(lx, axis=1, keepdims=True)
    i2 = jnp.min(jnp.where(lx == m2, e, NUM_CHILDREN), axis=1, keepdims=True)
    e2 = jnp.exp(m2 - m1)
    denom = 1.0 + e2
    w1 = 1.0 / denom
    w2 = e2 / denom
    o_ref[...] = jnp.where(e == i1, w1, 0.0) + jnp.where(e == i2, w2, 0.0)


def kernel(payload_tensor, W, b):
    b2 = b.reshape(1, NUM_CHILDREN)
    grid = (TOKENS // BT,)
    return pl.pallas_call(
        _fused_body,
        grid=grid,
        in_specs=[
            pl.BlockSpec((BT, D_MODEL), lambda i: (i, 0)),
            pl.BlockSpec((NUM_CHILDREN, D_MODEL), lambda i: (0, 0)),
            pl.BlockSpec((1, NUM_CHILDREN), lambda i: (0, 0)),
        ],
        out_specs=pl.BlockSpec((BT, NUM_CHILDREN), lambda i: (i, 0)),
        out_shape=jax.ShapeDtypeStruct((TOKENS, NUM_CHILDREN), jnp.float32),
        compiler_params=pltpu.CompilerParams(
            dimension_semantics=("arbitrary",),
        ),
    )(payload_tensor, W, b2)
